# Initial kernel scaffold; baseline (speedup 1.0000x reference)
import functools

import jax
import jax.numpy as jnp
from jax import lax
from jax.experimental import pallas as pl
from jax.experimental.pallas import tpu as pltpu
from jax.experimental.pallas import tpu_sc as plsc

NC = 2
NS = 16
NW = NC * NS
L = 16

D = 32
SCALE = float(D) ** 0.5
CHUNK = 128
B_TOTAL = 4096 * 200
PER_W = B_TOTAL // NW
NCHUNK = PER_W // CHUNK


def _emb_body(ids_hbm, table_hbm, out_hbm, idx_v, rows_v, gsem):
    wid = lax.axis_index("s") * NC + lax.axis_index("c")
    pltpu.sync_copy(ids_hbm.at[wid], idx_v)

    @pl.loop(0, NCHUNK)
    def _chunk(j):
        pltpu.async_copy(table_hbm.at[idx_v.at[j]], rows_v, gsem).wait()

        @pl.loop(0, CHUNK, unroll=8)
        def _scale(i):
            rows_v[i, 0:L] = rows_v[i, 0:L] * SCALE
            rows_v[i, L:D] = rows_v[i, L:D] * SCALE

        pltpu.sync_copy(rows_v, out_hbm.at[wid, j])


@jax.jit
def _emb_call(ids_flat, emb_var):
    mesh = plsc.VectorSubcoreMesh(
        core_axis_name="c", subcore_axis_name="s", num_cores=NC, num_subcores=NS
    )
    fn = pl.kernel(
        _emb_body,
        out_type=jax.ShapeDtypeStruct((NW, NCHUNK, CHUNK, D), jnp.float32),
        mesh=mesh,
        scratch_types=[
            pltpu.VMEM((NCHUNK, CHUNK), jnp.int32),
            pltpu.VMEM((CHUNK, D), jnp.float32),
            pltpu.SemaphoreType.DMA,
        ],
    )
    return fn(ids_flat, emb_var)


def kernel(ids, emb_var):
    b, s = ids.shape
    ids_flat = jnp.asarray(ids, jnp.int32).reshape(NW, NCHUNK, CHUNK)
    out = _emb_call(ids_flat, emb_var)
    return out.reshape(b, s, D)


# speedup vs baseline: 1.2569x; 1.2569x over previous
import functools

import jax
import jax.numpy as jnp
from jax import lax
from jax.experimental import pallas as pl
from jax.experimental.pallas import tpu as pltpu
from jax.experimental.pallas import tpu_sc as plsc

NC = 2
NS = 16
NW = NC * NS
L = 16

D = 32
SCALE = float(D) ** 0.5
CHUNK = 128
B_TOTAL = 4096 * 200
PER_W = B_TOTAL // NW
NCHUNK = PER_W // CHUNK


def _emb_body(ids_hbm, table_hbm, out_hbm, idx_v, rows_v, gsem):
    wid = lax.axis_index("s") * NC + lax.axis_index("c")
    pltpu.sync_copy(ids_hbm.at[wid], idx_v)

    @pl.loop(0, NCHUNK)
    def _chunk(j):
        pltpu.async_copy(table_hbm.at[idx_v.at[j]], rows_v, gsem).wait()

        @pl.loop(0, CHUNK, unroll=8)
        def _scale(i):
            rows_v[i, 0:L] = rows_v[i, 0:L] * SCALE
            rows_v[i, L:D] = rows_v[i, L:D] * SCALE

        pltpu.sync_copy(rows_v, out_hbm.at[wid, j])


@jax.jit
def _emb_call(ids_flat, emb_var):
    mesh = plsc.VectorSubcoreMesh(
        core_axis_name="c", subcore_axis_name="s", num_cores=NC, num_subcores=NS
    )
    fn = pl.kernel(
        _emb_body,
        out_type=jax.ShapeDtypeStruct((NW, NCHUNK, CHUNK, D), jnp.float32),
        mesh=mesh,
        scratch_types=[
            pltpu.VMEM((NCHUNK, CHUNK), jnp.int32),
            pltpu.VMEM((CHUNK, D), jnp.float32),
            pltpu.SemaphoreType.DMA,
        ],
        compiler_params=pltpu.CompilerParams(use_tc_tiling_on_sc=False),
    )
    return fn(ids_flat, emb_var)


def kernel(ids, emb_var):
    b, s = ids.shape
    ids_flat = jnp.asarray(ids, jnp.int32).reshape(NW, NCHUNK, CHUNK)
    out = _emb_call(ids_flat, emb_var)
    return out.reshape(b, s, D)


# speedup vs baseline: 1.4646x; 1.1653x over previous
import functools

import jax
import jax.numpy as jnp
from jax import lax
from jax.experimental import pallas as pl
from jax.experimental.pallas import tpu as pltpu
from jax.experimental.pallas import tpu_sc as plsc

NC = 2
NS = 16
NW = NC * NS
L = 16

D = 32
SCALE = float(D) ** 0.5
CHUNK = 128
B_TOTAL = 4096 * 200
PER_W = B_TOTAL // NW
NCHUNK = PER_W // CHUNK


NBUF = 4


def _emb_body(ids_hbm, table_hbm, out_hbm, idx_v, rows_v, *sems):
    gsems = sems[:NBUF]
    ssems = sems[NBUF:]
    wid = lax.axis_index("s") * NC + lax.axis_index("c")
    pltpu.sync_copy(ids_hbm.at[wid], idx_v)

    for b in range(NBUF):
        pltpu.async_copy(table_hbm.at[idx_v.at[b]], rows_v.at[b], gsems[b])

    @pl.loop(0, NCHUNK // NBUF)
    def _group(g):
        for b in range(NBUF):
            j = g * NBUF + b
            pltpu.make_async_copy(
                table_hbm.at[idx_v.at[j]], rows_v.at[b], gsems[b]
            ).wait()

            bp = (b - 1) % NBUF
            jr = j - 1 + NBUF

            @pl.when(jnp.logical_and(j >= 1, jr < NCHUNK))
            def _refill():
                pltpu.make_async_copy(
                    rows_v.at[bp], out_hbm.at[wid, jr - NBUF], ssems[bp]
                ).wait()
                pltpu.async_copy(
                    table_hbm.at[idx_v.at[jr]], rows_v.at[bp], gsems[bp]
                )

            @pl.loop(0, CHUNK, unroll=8)
            def _scale(i):
                rows_v[b, i, 0:L] = rows_v[b, i, 0:L] * SCALE
                rows_v[b, i, L:D] = rows_v[b, i, L:D] * SCALE

            pltpu.async_copy(rows_v.at[b], out_hbm.at[wid, j], ssems[b])

    for k in range(NCHUNK - NBUF, NCHUNK):
        b = k % NBUF
        pltpu.make_async_copy(rows_v.at[b], out_hbm.at[wid, k], ssems[b]).wait()


@jax.jit
def _emb_call(ids_flat, emb_var):
    mesh = plsc.VectorSubcoreMesh(
        core_axis_name="c", subcore_axis_name="s", num_cores=NC, num_subcores=NS
    )
    fn = pl.kernel(
        _emb_body,
        out_type=jax.ShapeDtypeStruct((NW, NCHUNK, CHUNK, D), jnp.float32),
        mesh=mesh,
        scratch_types=[
            pltpu.VMEM((NCHUNK, CHUNK), jnp.int32),
            pltpu.VMEM((NBUF, CHUNK, D), jnp.float32),
        ]
        + [pltpu.SemaphoreType.DMA] * (2 * NBUF),
        compiler_params=pltpu.CompilerParams(use_tc_tiling_on_sc=False),
    )
    return fn(ids_flat, emb_var)


def kernel(ids, emb_var):
    b, s = ids.shape
    ids_flat = jnp.asarray(ids, jnp.int32).reshape(NW, NCHUNK, CHUNK)
    out = _emb_call(ids_flat, emb_var)
    return out.reshape(b, s, D)
